# Initial kernel scaffold; baseline (speedup 1.0000x reference)
import jax
import jax.numpy as jnp
from jax.experimental import pallas as pl


def kernel(idx, logits_table):
    raise NotImplementedError("write your pallas kernel here")


# speedup vs baseline: 1.0014x; 1.0014x over previous
import functools

import jax
import jax.numpy as jnp
from jax import lax
from jax.experimental import pallas as pl
from jax.experimental.pallas import tpu as pltpu
from jax.experimental.pallas import tpu_sc as plsc

VOCAB = 1000
NC = 2
NS = 16
NW = NC * NS
CHUNK = 40


def _gather_body(nchunk, idx_hbm, table_hbm, out_hbm, idx_v, buf, sem):
    c = lax.axis_index("c")
    s = lax.axis_index("s")
    wid = s * NC + c
    rows_per_w = nchunk * CHUNK
    pltpu.sync_copy(idx_hbm.at[wid], idx_v)

    def step(i, carry):
        pltpu.async_copy(table_hbm.at[idx_v.at[i]], buf, sem).wait()
        pltpu.sync_copy(buf, out_hbm.at[pl.ds(wid * rows_per_w + i * CHUNK, CHUNK)])
        return carry

    lax.fori_loop(0, nchunk, step, 0)


def kernel(idx, logits_table):
    B, L = idx.shape
    N = B * L
    assert N % (NW * CHUNK) == 0
    nchunk = N // (NW * CHUNK)
    idx3 = idx.reshape(NW, nchunk, CHUNK).astype(jnp.int32)

    mesh = plsc.VectorSubcoreMesh(core_axis_name="c", subcore_axis_name="s")
    k = pl.kernel(
        functools.partial(_gather_body, nchunk),
        out_type=jax.ShapeDtypeStruct((N, VOCAB), jnp.float32),
        mesh=mesh,
        scratch_types=[
            pltpu.VMEM((nchunk, CHUNK), jnp.int32),
            pltpu.VMEM((CHUNK, VOCAB), jnp.float32),
            pltpu.SemaphoreType.DMA,
        ],
        compiler_params=pltpu.CompilerParams(use_tc_tiling_on_sc=False),
    )
    out = k(idx3, logits_table)
    return out.reshape(B, L, VOCAB)


# speedup vs baseline: 1.0340x; 1.0326x over previous
import functools

import jax
import jax.numpy as jnp
from jax import lax
from jax.experimental import pallas as pl
from jax.experimental.pallas import tpu as pltpu
from jax.experimental.pallas import tpu_sc as plsc

VOCAB = 1000
NC = 2
NS = 16
NW = NC * NS
CHUNK = 50


def _gather_body(nchunk, idx_hbm, table_hbm, out_hbm, idx_v, buf0, buf1,
                 sem0, sem1):
    c = lax.axis_index("c")
    s = lax.axis_index("s")
    wid = s * NC + c
    rows_per_w = nchunk * CHUNK
    out_base = wid * rows_per_w
    bufs = (buf0, buf1)
    sems = (sem0, sem1)

    pltpu.sync_copy(idx_hbm.at[wid], idx_v)

    pltpu.async_copy(table_hbm.at[idx_v.at[0]], buf0, sem0)

    def step(i, carry):
        for b in range(2):
            cur = 2 * i + b
            nxt = cur + 1
            pltpu.make_async_copy(
                table_hbm.at[idx_v.at[cur]], bufs[b], sems[b]).wait()
            @pl.when(nxt < nchunk)
            def _():
                pltpu.async_copy(
                    table_hbm.at[idx_v.at[nxt]], bufs[1 - b], sems[1 - b])
            pltpu.sync_copy(
                bufs[b], out_hbm.at[pl.ds(out_base + cur * CHUNK, CHUNK)])
        return carry

    lax.fori_loop(0, nchunk // 2, step, 0)


def kernel(idx, logits_table):
    B, L = idx.shape
    N = B * L
    assert N % (NW * CHUNK) == 0
    nchunk = N // (NW * CHUNK)
    assert nchunk % 2 == 0
    idx3 = idx.reshape(NW, nchunk, CHUNK).astype(jnp.int32)

    mesh = plsc.VectorSubcoreMesh(core_axis_name="c", subcore_axis_name="s")
    k = pl.kernel(
        functools.partial(_gather_body, nchunk),
        out_type=jax.ShapeDtypeStruct((N, VOCAB), jnp.float32),
        mesh=mesh,
        scratch_types=[
            pltpu.VMEM((nchunk, CHUNK), jnp.int32),
            pltpu.VMEM((CHUNK, VOCAB), jnp.float32),
            pltpu.VMEM((CHUNK, VOCAB), jnp.float32),
            pltpu.SemaphoreType.DMA,
            pltpu.SemaphoreType.DMA,
        ],
        compiler_params=pltpu.CompilerParams(use_tc_tiling_on_sc=False),
    )
    out = k(idx3, logits_table)
    return out.reshape(B, L, VOCAB)


# speedup vs baseline: 1.1570x; 1.1189x over previous
import functools

import jax
import jax.numpy as jnp
from jax import lax
from jax.experimental import pallas as pl
from jax.experimental.pallas import tpu as pltpu
from jax.experimental.pallas import tpu_sc as plsc

VOCAB = 1000
NC = 2
NS = 16
NW = NC * NS
CHUNK = 25


def _gather_body(nchunk, idx_hbm, table_hbm, out_hbm, idx_v, table_sh,
                 buf0, buf1, sem0, sem1):
    c = lax.axis_index("c")
    s = lax.axis_index("s")
    wid = s * NC + c
    rows_per_w = nchunk * CHUNK
    out_base = wid * rows_per_w
    bufs = (buf0, buf1)
    sems = (sem0, sem1)

    @pl.when(s < 8)
    def _():
        pltpu.sync_copy(table_hbm.at[pl.ds(s * 125, 125)],
                        table_sh.at[pl.ds(s * 125, 125)])

    pltpu.sync_copy(idx_hbm.at[wid], idx_v)
    plsc.subcore_barrier()

    pltpu.async_copy(table_sh.at[idx_v.at[0]], buf0, sem0)

    def step(i, carry):
        for b in range(2):
            cur = 2 * i + b
            nxt = cur + 1
            pltpu.make_async_copy(
                table_sh.at[idx_v.at[cur]], bufs[b], sems[b]).wait()
            @pl.when(nxt < nchunk)
            def _():
                pltpu.async_copy(
                    table_sh.at[idx_v.at[nxt]], bufs[1 - b], sems[1 - b])
            pltpu.sync_copy(
                bufs[b], out_hbm.at[pl.ds(out_base + cur * CHUNK, CHUNK)])
        return carry

    lax.fori_loop(0, nchunk // 2, step, 0)


def kernel(idx, logits_table):
    B, L = idx.shape
    N = B * L
    assert N % (NW * CHUNK) == 0
    nchunk = N // (NW * CHUNK)
    assert nchunk % 2 == 0
    idx3 = idx.reshape(NW, nchunk, CHUNK).astype(jnp.int32)

    mesh = plsc.VectorSubcoreMesh(core_axis_name="c", subcore_axis_name="s")
    k = pl.kernel(
        functools.partial(_gather_body, nchunk),
        out_type=jax.ShapeDtypeStruct((N, VOCAB), jnp.float32),
        mesh=mesh,
        scratch_types=[
            pltpu.VMEM((nchunk, CHUNK), jnp.int32),
            pltpu.VMEM_SHARED((VOCAB, VOCAB), jnp.float32),
            pltpu.VMEM((CHUNK, VOCAB), jnp.float32),
            pltpu.VMEM((CHUNK, VOCAB), jnp.float32),
            pltpu.SemaphoreType.DMA,
            pltpu.SemaphoreType.DMA,
        ],
        compiler_params=pltpu.CompilerParams(use_tc_tiling_on_sc=False),
    )
    out = k(idx3, logits_table)
    return out.reshape(B, L, VOCAB)
